# baseline (speedup 1.0000x reference)
import functools

import jax
import jax.numpy as jnp
from jax import lax
from jax.experimental import pallas as pl
from jax.experimental.pallas import tpu as pltpu
from jax.experimental.pallas import tpu_sc as plsc

NC = 2
NS = 16
NW = NC * NS
K = 128
LANES = 16

_mesh = lambda: plsc.VectorSubcoreMesh(core_axis_name="c", subcore_axis_name="s")


def _fill1d(ref, n, val):
    v = jnp.full((LANES,), val, ref.dtype)

    def body(i, _):
        ref[pl.ds(pl.multiple_of(i * LANES, LANES), LANES)] = v
        return ()

    lax.fori_loop(0, n // LANES, body, ())


def _zero2d(ref, rows, cols):
    v = jnp.zeros((LANES,), ref.dtype)

    def body(i, _):
        r = i // (cols // LANES)
        q = i % (cols // LANES)
        ref[r, pl.ds(pl.multiple_of(q * LANES, LANES), LANES)] = v
        return ()

    lax.fori_loop(0, rows * (cols // LANES), body, ())


def _make_deg(NA, TOTCH):
    RPT = NA // NS
    CHW = TOTCH // NW

    @functools.partial(
        pl.kernel,
        out_type=jax.ShapeDtypeStruct((NC, NA), jnp.float32),
        mesh=_mesh(),
        scratch_types=[
            pltpu.VMEM((CHW, K), jnp.int32),
            pltpu.VMEM((K,), jnp.float32),
            pltpu.VMEM((RPT,), jnp.float32),
            pltpu.VMEM_SHARED((NA,), jnp.float32),
        ],
    )
    def deg_kernel(dst_hbm, out_hbm, dst_v, ones_v, z_v, acc_sh):
        c = lax.axis_index("c")
        s = lax.axis_index("s")
        wid = s * NC + c
        _fill1d(z_v, RPT, 0.0)
        _fill1d(ones_v, K, 1.0)
        pltpu.sync_copy(z_v, acc_sh.at[pl.ds(s * RPT, RPT)])
        plsc.subcore_barrier()
        pltpu.sync_copy(dst_hbm.at[pl.ds(wid * CHW, CHW)], dst_v)

        def body(j, _):
            pltpu.sync_copy(ones_v, acc_sh.at[dst_v.at[j]], add=True)
            return ()

        lax.fori_loop(0, CHW, body, ())
        plsc.subcore_barrier()
        pltpu.sync_copy(acc_sh.at[pl.ds(s * RPT, RPT)],
                        out_hbm.at[c, pl.ds(s * RPT, RPT)])

    return deg_kernel


def _make_segsum(N, H, NA, TOTCH, C0):
    RPT = NA // NS
    ZCH = RPT // K
    CPT0 = C0 // NS
    CPT1 = (TOTCH - C0) // NS

    @functools.partial(
        pl.kernel,
        out_type=jax.ShapeDtypeStruct((NC, NA, H), jnp.float32),
        mesh=_mesh(),
        scratch_types=[
            pltpu.VMEM((2, K), jnp.int32),
            pltpu.VMEM((2, K), jnp.int32),
            pltpu.VMEM((K, H), jnp.float32),
            pltpu.VMEM((K, H), jnp.float32),
            pltpu.VMEM_SHARED((NA, H), jnp.float32),
            pltpu.SemaphoreType.DMA,
            pltpu.SemaphoreType.DMA,
            pltpu.SemaphoreType.DMA,
        ],
    )
    def segsum_kernel(g_hbm, src_hbm, dst_hbm, out_hbm,
                      i0, i1, buf0, buf1, acc_sh, semg0, semg1, sems0):
        c = lax.axis_index("c")
        s = lax.axis_index("s")
        _zero2d(buf0, K, H)

        def zb(r, _):
            pltpu.sync_copy(buf0, acc_sh.at[pl.ds(s * RPT + r * K, K)])
            return ()

        lax.fori_loop(0, ZCH, zb, ())
        plsc.subcore_barrier()

        def load_idx(j, iv):
            pltpu.sync_copy(src_hbm.at[j], iv.at[0])
            pltpu.sync_copy(dst_hbm.at[j], iv.at[1])

        def core_run(base0, cpt):
            if cpt == 0:
                return
            base = base0 + s * cpt
            last = base + cpt - 1
            load_idx(base, i0)
            pltpu.async_copy(g_hbm.at[i0.at[0]], buf0, semg0)
            load_idx(base + 1, i1)

            def body(jj, _):
                j2 = jnp.minimum(base + jj * 2 + 2, last)
                j3 = jnp.minimum(base + jj * 2 + 3, last)
                pltpu.async_copy(g_hbm.at[i1.at[0]], buf1, semg1)
                pltpu.make_async_copy(g_hbm.at[i0.at[0]], buf0, semg0).wait()
                s0 = pltpu.async_copy(buf0, acc_sh.at[i0.at[1]], sems0,
                                      add=True)
                pltpu.make_async_copy(g_hbm.at[i1.at[0]], buf1, semg1).wait()
                s0.wait()
                load_idx(j2, i0)
                pltpu.async_copy(g_hbm.at[i0.at[0]], buf0, semg0)
                pltpu.sync_copy(buf1, acc_sh.at[i1.at[1]], add=True)
                load_idx(j3, i1)
                return ()

            lax.fori_loop(0, cpt // 2, body, ())
            pltpu.make_async_copy(g_hbm.at[i0.at[0]], buf0, semg0).wait()

        @pl.when(c == 0)
        def _():
            core_run(0, CPT0)

        @pl.when(c == 1)
        def _():
            core_run(C0, CPT1)

        plsc.subcore_barrier()

        def wb(r, _):
            pltpu.sync_copy(acc_sh.at[pl.ds(s * RPT + r * K, K)],
                            out_hbm.at[c, pl.ds(s * RPT + r * K, K)])
            return ()

        lax.fori_loop(0, ZCH, wb, ())

    return segsum_kernel


def _make_decode(N, H, TOTCH, C0):
    NQ = H // LANES
    CPT0 = C0 // NS
    CPT1 = (TOTCH - C0) // NS
    CPTM = max(CPT0, CPT1, 2)

    @functools.partial(
        pl.kernel,
        out_type=jax.ShapeDtypeStruct((TOTCH * K,), jnp.float32),
        mesh=_mesh(),
        scratch_types=[
            pltpu.VMEM((CPTM, K), jnp.int32),
            pltpu.VMEM((CPTM, K), jnp.int32),
            pltpu.VMEM((K, H), jnp.float32),
            pltpu.VMEM((K, H), jnp.float32),
            pltpu.VMEM((K, H), jnp.float32),
            pltpu.VMEM((K, H), jnp.float32),
            pltpu.VMEM((CPTM * K,), jnp.float32),
            pltpu.SemaphoreType.DMA,
            pltpu.SemaphoreType.DMA,
        ],
    )
    def decode_kernel(z_hbm, src_hbm, dst_hbm, out_hbm,
                      src_v, dst_v, a0, b0, a1, b1, sc_v, sem0, sem1):
        c = lax.axis_index("c")
        s = lax.axis_index("s")
        lane = lax.iota(jnp.int32, LANES)
        bfly = [lane ^ sh for sh in (8, 4, 2, 1)]

        def chunk_dot(a, b, j):
            def grp(g, _):
                vec = jnp.zeros((LANES,), jnp.float32)
                for i in range(LANES):
                    e = g * LANES + i
                    acc = a[e, pl.ds(0, LANES)] * b[e, pl.ds(0, LANES)]
                    for q in range(1, NQ):
                        acc = acc + (a[e, pl.ds(q * LANES, LANES)]
                                     * b[e, pl.ds(q * LANES, LANES)])
                    for idx in bfly:
                        acc = acc + acc.at[idx].get(mode="promise_in_bounds")
                    vec = jnp.where(lane == i, acc, vec)
                base = pl.multiple_of(j * K + g * LANES, LANES)
                sc_v[pl.ds(base, LANES)] = vec
                return ()

            lax.fori_loop(0, K // LANES, grp, ())

        def gather_pair(j, a, b, sem):
            pltpu.async_copy(z_hbm.at[src_v.at[j]], a, sem)
            pltpu.async_copy(z_hbm.at[dst_v.at[j]], b, sem)

        def wait_pair(j, a, b, sem):
            pltpu.make_async_copy(z_hbm.at[src_v.at[j]], a, sem).wait()
            pltpu.make_async_copy(z_hbm.at[dst_v.at[j]], b, sem).wait()

        def core_run(base0, cpt):
            if cpt == 0:
                return
            base = base0 + s * cpt
            pltpu.sync_copy(src_hbm.at[pl.ds(base, cpt)],
                            src_v.at[pl.ds(0, cpt)])
            pltpu.sync_copy(dst_hbm.at[pl.ds(base, cpt)],
                            dst_v.at[pl.ds(0, cpt)])
            gather_pair(0, a0, b0, sem0)

            def body(jj, _):
                j0 = jj * 2
                j1 = j0 + 1
                j2 = jnp.minimum(j0 + 2, cpt - 1)
                gather_pair(j1, a1, b1, sem1)
                wait_pair(j0, a0, b0, sem0)
                chunk_dot(a0, b0, j0)
                gather_pair(j2, a0, b0, sem0)
                wait_pair(j1, a1, b1, sem1)
                chunk_dot(a1, b1, j1)
                return ()

            lax.fori_loop(0, cpt // 2, body, ())
            wait_pair(0, a0, b0, sem0)
            pltpu.sync_copy(sc_v.at[pl.ds(0, cpt * K)],
                            out_hbm.at[pl.ds(base * K, cpt * K)])

        @pl.when(c == 0)
        def _():
            core_run(0, CPT0)

        @pl.when(c == 1)
        def _():
            core_run(C0, CPT1)

    return decode_kernel


def _make_enc1(N, H, BM):
    def body(deg_ref, x_ref, w_ref, out_ref):
        h = jnp.dot(x_ref[...], w_ref[...], preferred_element_type=jnp.float32)
        dinv = lax.rsqrt(deg_ref[:, 0] + deg_ref[:, 1] + 1.0)
        out_ref[...] = h * dinv[:, None]

    return pl.pallas_call(
        body,
        grid=(N // BM,),
        in_specs=[
            pl.BlockSpec((BM, 2), lambda i: (i, 0)),
            pl.BlockSpec((BM, N), lambda i: (i, 0)),
            pl.BlockSpec((N, H), lambda i: (0, 0)),
        ],
        out_specs=pl.BlockSpec((BM, H), lambda i: (i, 0)),
        out_shape=jax.ShapeDtypeStruct((N, H), jnp.float32),
    )


def _make_enc2(N, H, NA, BM):
    def body(deg_ref, s_ref, g_ref, b_ref, w_ref, out_ref):
        dinv = lax.rsqrt(deg_ref[:, 0] + deg_ref[:, 1] + 1.0)[:, None]
        z1 = dinv * (s_ref[0] + s_ref[1] + g_ref[...]) + b_ref[...]
        z1 = jnp.maximum(z1, 0.0)
        h2 = jnp.dot(z1, w_ref[...], preferred_element_type=jnp.float32)
        out_ref[...] = h2 * dinv

    return pl.pallas_call(
        body,
        grid=(N // BM,),
        in_specs=[
            pl.BlockSpec((BM, 2), lambda i: (i, 0)),
            pl.BlockSpec((2, BM, H), lambda i: (0, i, 0)),
            pl.BlockSpec((BM, H), lambda i: (i, 0)),
            pl.BlockSpec((1, H), lambda i: (0, 0)),
            pl.BlockSpec((H, H), lambda i: (0, 0)),
        ],
        out_specs=pl.BlockSpec((BM, H), lambda i: (i, 0)),
        out_shape=jax.ShapeDtypeStruct((N, H), jnp.float32),
    )


def _make_fin(N, H, NA, BM):
    def body(deg_ref, s_ref, g_ref, b_ref, out_ref):
        dinv = lax.rsqrt(deg_ref[:, 0] + deg_ref[:, 1] + 1.0)[:, None]
        out_ref[...] = dinv * (s_ref[0] + s_ref[1] + g_ref[...]) + b_ref[...]

    return pl.pallas_call(
        body,
        grid=(N // BM,),
        in_specs=[
            pl.BlockSpec((BM, 2), lambda i: (i, 0)),
            pl.BlockSpec((2, BM, H), lambda i: (0, i, 0)),
            pl.BlockSpec((BM, H), lambda i: (i, 0)),
            pl.BlockSpec((1, H), lambda i: (0, 0)),
        ],
        out_specs=pl.BlockSpec((BM, H), lambda i: (i, 0)),
        out_shape=jax.ShapeDtypeStruct((N, H), jnp.float32),
    )


C0_SEG1 = 640
C0_SEG2 = 640
C0_DEC = 640


def kernel(x, edge_index, W1, b1, W2, b2):
    N = x.shape[0]
    H = W1.shape[1]
    E = edge_index.shape[1]
    TOTCH = -(-E // (NW * K)) * NW
    EP = TOTCH * K
    NA = -(-(N + 1) // (NS * K)) * (NS * K)
    BM = 400

    src = edge_index[0]
    dst = edge_index[1]
    pad = EP - E
    pidx = jnp.arange(pad, dtype=src.dtype)
    src_p = jnp.concatenate([src, pidx % N])
    dst_seg = jnp.concatenate([dst, N + pidx % (NA - N)])
    dst_dec = jnp.concatenate([dst, pidx % N])
    src2 = src_p.reshape(TOTCH, K)
    dst2s = dst_seg.reshape(TOTCH, K)
    dst2d = dst_dec.reshape(TOTCH, K)
    degp = _make_deg(NA, TOTCH)(dst2s)
    degp2 = degp[:, :N].T
    b1r = b1.reshape(1, H)
    b2r = b2.reshape(1, H)
    BM2 = 1000

    g1 = _make_enc1(N, H, BM)(degp2, x, W1)
    s1 = _make_segsum(N, H, NA, TOTCH, C0_SEG1)(g1, src2, dst2s)
    g2 = _make_enc2(N, H, NA, BM2)(degp2, s1, g1, b1r, W2)
    s2 = _make_segsum(N, H, NA, TOTCH, C0_SEG2)(g2, src2, dst2s)
    z2 = _make_fin(N, H, NA, BM2)(degp2, s2, g2, b2r)
    scores = _make_decode(N, H, TOTCH, C0_DEC)(z2, src2, dst2d)
    return scores[:E]


# speedup vs baseline: 1.0035x; 1.0035x over previous
import functools

import jax
import jax.numpy as jnp
from jax import lax
from jax.experimental import pallas as pl
from jax.experimental.pallas import tpu as pltpu
from jax.experimental.pallas import tpu_sc as plsc

NC = 2
NS = 16
NW = NC * NS
K = 128
LANES = 16

_mesh = lambda: plsc.VectorSubcoreMesh(core_axis_name="c", subcore_axis_name="s")


def _fill1d(ref, n, val):
    v = jnp.full((LANES,), val, ref.dtype)

    def body(i, _):
        ref[pl.ds(pl.multiple_of(i * LANES, LANES), LANES)] = v
        return ()

    lax.fori_loop(0, n // LANES, body, ())


def _zero2d(ref, rows, cols):
    v = jnp.zeros((LANES,), ref.dtype)

    def body(i, _):
        r = i // (cols // LANES)
        q = i % (cols // LANES)
        ref[r, pl.ds(pl.multiple_of(q * LANES, LANES), LANES)] = v
        return ()

    lax.fori_loop(0, rows * (cols // LANES), body, ())


def _make_deg(NA, TOTCH):
    RPT = NA // NS
    CHW = TOTCH // NW

    @functools.partial(
        pl.kernel,
        out_type=jax.ShapeDtypeStruct((NC, NA), jnp.float32),
        mesh=_mesh(),
        scratch_types=[
            pltpu.VMEM((CHW, K), jnp.int32),
            pltpu.VMEM((K,), jnp.float32),
            pltpu.VMEM((RPT,), jnp.float32),
            pltpu.VMEM_SHARED((NA,), jnp.float32),
        ],
    )
    def deg_kernel(dst_hbm, out_hbm, dst_v, ones_v, z_v, acc_sh):
        c = lax.axis_index("c")
        s = lax.axis_index("s")
        wid = s * NC + c
        _fill1d(z_v, RPT, 0.0)
        _fill1d(ones_v, K, 1.0)
        pltpu.sync_copy(z_v, acc_sh.at[pl.ds(s * RPT, RPT)])
        plsc.subcore_barrier()
        pltpu.sync_copy(dst_hbm.at[pl.ds(wid * CHW, CHW)], dst_v)

        def body(j, _):
            pltpu.sync_copy(ones_v, acc_sh.at[dst_v.at[j]], add=True)
            return ()

        lax.fori_loop(0, CHW, body, ())
        plsc.subcore_barrier()
        pltpu.sync_copy(acc_sh.at[pl.ds(s * RPT, RPT)],
                        out_hbm.at[c, pl.ds(s * RPT, RPT)])

    return deg_kernel


def _make_segsum(N, H, NA, TOTCH, C0):
    RPT = NA // NS
    ZCH = RPT // K
    CPT0 = C0 // NS
    CPT1 = (TOTCH - C0) // NS

    @functools.partial(
        pl.kernel,
        out_type=jax.ShapeDtypeStruct((NC, NA, H), jnp.float32),
        mesh=_mesh(),
        scratch_types=[
            pltpu.VMEM((2, K), jnp.int32),
            pltpu.VMEM((2, K), jnp.int32),
            pltpu.VMEM((K, H), jnp.float32),
            pltpu.VMEM((K, H), jnp.float32),
            pltpu.VMEM_SHARED((NA, H), jnp.float32),
            pltpu.SemaphoreType.DMA,
            pltpu.SemaphoreType.DMA,
            pltpu.SemaphoreType.DMA,
        ],
    )
    def segsum_kernel(g_hbm, src_hbm, dst_hbm, out_hbm,
                      i0, i1, buf0, buf1, acc_sh, semg0, semg1, sems0):
        c = lax.axis_index("c")
        s = lax.axis_index("s")
        _zero2d(buf0, K, H)

        def zb(r, _):
            pltpu.sync_copy(buf0, acc_sh.at[pl.ds(s * RPT + r * K, K)])
            return ()

        lax.fori_loop(0, ZCH, zb, ())
        plsc.subcore_barrier()

        def load_idx(j, iv):
            pltpu.sync_copy(src_hbm.at[j], iv.at[0])
            pltpu.sync_copy(dst_hbm.at[j], iv.at[1])

        def core_run(base0, cpt):
            if cpt == 0:
                return
            base = base0 + s * cpt
            last = base + cpt - 1
            load_idx(base, i0)
            pltpu.async_copy(g_hbm.at[i0.at[0]], buf0, semg0)
            load_idx(base + 1, i1)

            def body(jj, _):
                j2 = jnp.minimum(base + jj * 2 + 2, last)
                j3 = jnp.minimum(base + jj * 2 + 3, last)
                pltpu.async_copy(g_hbm.at[i1.at[0]], buf1, semg1)
                pltpu.make_async_copy(g_hbm.at[i0.at[0]], buf0, semg0).wait()
                s0 = pltpu.async_copy(buf0, acc_sh.at[i0.at[1]], sems0,
                                      add=True)
                pltpu.make_async_copy(g_hbm.at[i1.at[0]], buf1, semg1).wait()
                s0.wait()
                load_idx(j2, i0)
                pltpu.async_copy(g_hbm.at[i0.at[0]], buf0, semg0)
                pltpu.sync_copy(buf1, acc_sh.at[i1.at[1]], add=True)
                load_idx(j3, i1)
                return ()

            lax.fori_loop(0, cpt // 2, body, ())
            pltpu.make_async_copy(g_hbm.at[i0.at[0]], buf0, semg0).wait()

        @pl.when(c == 0)
        def _():
            core_run(0, CPT0)

        @pl.when(c == 1)
        def _():
            core_run(C0, CPT1)

        plsc.subcore_barrier()

        def wb(r, _):
            pltpu.sync_copy(acc_sh.at[pl.ds(s * RPT + r * K, K)],
                            out_hbm.at[c, pl.ds(s * RPT + r * K, K)])
            return ()

        lax.fori_loop(0, ZCH, wb, ())

    return segsum_kernel


def _make_decode(N, H, TOTCH, C0):
    NQ = H // LANES
    CPT0 = C0 // NS
    CPT1 = (TOTCH - C0) // NS
    CPTM = max(CPT0, CPT1, 2)

    @functools.partial(
        pl.kernel,
        out_type=jax.ShapeDtypeStruct((TOTCH * K,), jnp.float32),
        mesh=_mesh(),
        scratch_types=[
            pltpu.VMEM((CPTM, K), jnp.int32),
            pltpu.VMEM((CPTM, K), jnp.int32),
            pltpu.VMEM((K, H), jnp.float32),
            pltpu.VMEM((K, H), jnp.float32),
            pltpu.VMEM((K, H), jnp.float32),
            pltpu.VMEM((K, H), jnp.float32),
            pltpu.VMEM((CPTM * K,), jnp.float32),
            pltpu.SemaphoreType.DMA,
            pltpu.SemaphoreType.DMA,
        ],
    )
    def decode_kernel(z_hbm, src_hbm, dst_hbm, out_hbm,
                      src_v, dst_v, a0, b0, a1, b1, sc_v, sem0, sem1):
        c = lax.axis_index("c")
        s = lax.axis_index("s")
        lane = lax.iota(jnp.int32, LANES)
        bfly = [lane ^ sh for sh in (8, 4, 2, 1)]

        def chunk_dot(a, b, j):
            def grp(g, _):
                vec = jnp.zeros((LANES,), jnp.float32)
                for i in range(LANES):
                    e = g * LANES + i
                    acc = a[e, pl.ds(0, LANES)] * b[e, pl.ds(0, LANES)]
                    for q in range(1, NQ):
                        acc = acc + (a[e, pl.ds(q * LANES, LANES)]
                                     * b[e, pl.ds(q * LANES, LANES)])
                    for idx in bfly:
                        acc = acc + acc.at[idx].get(mode="promise_in_bounds")
                    vec = jnp.where(lane == i, acc, vec)
                base = pl.multiple_of(j * K + g * LANES, LANES)
                sc_v[pl.ds(base, LANES)] = vec
                return ()

            lax.fori_loop(0, K // LANES, grp, ())

        def gather_pair(j, a, b, sem):
            pltpu.async_copy(z_hbm.at[src_v.at[j]], a, sem)
            pltpu.async_copy(z_hbm.at[dst_v.at[j]], b, sem)

        def wait_pair(j, a, b, sem):
            pltpu.make_async_copy(z_hbm.at[src_v.at[j]], a, sem).wait()
            pltpu.make_async_copy(z_hbm.at[dst_v.at[j]], b, sem).wait()

        def core_run(base0, cpt):
            if cpt == 0:
                return
            base = base0 + s * cpt
            pltpu.sync_copy(src_hbm.at[pl.ds(base, cpt)],
                            src_v.at[pl.ds(0, cpt)])
            pltpu.sync_copy(dst_hbm.at[pl.ds(base, cpt)],
                            dst_v.at[pl.ds(0, cpt)])
            gather_pair(0, a0, b0, sem0)

            def body(jj, _):
                j0 = jj * 2
                j1 = j0 + 1
                j2 = jnp.minimum(j0 + 2, cpt - 1)
                gather_pair(j1, a1, b1, sem1)
                wait_pair(j0, a0, b0, sem0)
                chunk_dot(a0, b0, j0)
                gather_pair(j2, a0, b0, sem0)
                wait_pair(j1, a1, b1, sem1)
                chunk_dot(a1, b1, j1)
                return ()

            lax.fori_loop(0, cpt // 2, body, ())
            wait_pair(0, a0, b0, sem0)
            pltpu.sync_copy(sc_v.at[pl.ds(0, cpt * K)],
                            out_hbm.at[pl.ds(base * K, cpt * K)])

        @pl.when(c == 0)
        def _():
            core_run(0, CPT0)

        @pl.when(c == 1)
        def _():
            core_run(C0, CPT1)

    return decode_kernel


def _make_enc1(N, H, BM):
    def body(deg_ref, x_ref, w_ref, out_ref):
        h = jnp.dot(x_ref[...], w_ref[...], preferred_element_type=jnp.float32)
        dinv = lax.rsqrt(deg_ref[:, 0] + deg_ref[:, 1] + 1.0)
        out_ref[...] = h * dinv[:, None]

    return pl.pallas_call(
        body,
        grid=(N // BM,),
        in_specs=[
            pl.BlockSpec((BM, 2), lambda i: (i, 0)),
            pl.BlockSpec((BM, N), lambda i: (i, 0)),
            pl.BlockSpec((N, H), lambda i: (0, 0)),
        ],
        out_specs=pl.BlockSpec((BM, H), lambda i: (i, 0)),
        out_shape=jax.ShapeDtypeStruct((N, H), jnp.float32),
    )


def _make_enc2(N, H, NA, BM):
    def body(deg_ref, s_ref, g_ref, b_ref, w_ref, out_ref):
        dinv = lax.rsqrt(deg_ref[:, 0] + deg_ref[:, 1] + 1.0)[:, None]
        z1 = dinv * (s_ref[0] + s_ref[1] + g_ref[...]) + b_ref[...]
        z1 = jnp.maximum(z1, 0.0)
        h2 = jnp.dot(z1, w_ref[...], preferred_element_type=jnp.float32)
        out_ref[...] = h2 * dinv

    return pl.pallas_call(
        body,
        grid=(N // BM,),
        in_specs=[
            pl.BlockSpec((BM, 2), lambda i: (i, 0)),
            pl.BlockSpec((2, BM, H), lambda i: (0, i, 0)),
            pl.BlockSpec((BM, H), lambda i: (i, 0)),
            pl.BlockSpec((1, H), lambda i: (0, 0)),
            pl.BlockSpec((H, H), lambda i: (0, 0)),
        ],
        out_specs=pl.BlockSpec((BM, H), lambda i: (i, 0)),
        out_shape=jax.ShapeDtypeStruct((N, H), jnp.float32),
    )


def _make_fin(N, H, NA, BM):
    def body(deg_ref, s_ref, g_ref, b_ref, out_ref):
        dinv = lax.rsqrt(deg_ref[:, 0] + deg_ref[:, 1] + 1.0)[:, None]
        out_ref[...] = dinv * (s_ref[0] + s_ref[1] + g_ref[...]) + b_ref[...]

    return pl.pallas_call(
        body,
        grid=(N // BM,),
        in_specs=[
            pl.BlockSpec((BM, 2), lambda i: (i, 0)),
            pl.BlockSpec((2, BM, H), lambda i: (0, i, 0)),
            pl.BlockSpec((BM, H), lambda i: (i, 0)),
            pl.BlockSpec((1, H), lambda i: (0, 0)),
        ],
        out_specs=pl.BlockSpec((BM, H), lambda i: (i, 0)),
        out_shape=jax.ShapeDtypeStruct((N, H), jnp.float32),
    )


C0_SEG1 = 640
C0_SEG2 = 640
C0_DEC = 640


def kernel(x, edge_index, W1, b1, W2, b2):
    N = x.shape[0]
    H = W1.shape[1]
    E = edge_index.shape[1]
    TOTCH = -(-E // (NW * K)) * NW
    EP = TOTCH * K
    NA = -(-(N + 1) // (NS * K)) * (NS * K)
    BM = 200

    src = edge_index[0]
    dst = edge_index[1]
    pad = EP - E
    pidx = jnp.arange(pad, dtype=src.dtype)
    src_p = jnp.concatenate([src, pidx % N])
    dst_seg = jnp.concatenate([dst, N + pidx % (NA - N)])
    dst_dec = jnp.concatenate([dst, pidx % N])
    src2 = src_p.reshape(TOTCH, K)
    dst2s = dst_seg.reshape(TOTCH, K)
    dst2d = dst_dec.reshape(TOTCH, K)
    degp = _make_deg(NA, TOTCH)(dst2s)
    degp2 = degp[:, :N].T
    b1r = b1.reshape(1, H)
    b2r = b2.reshape(1, H)
    BM2 = 1000

    g1 = _make_enc1(N, H, BM)(degp2, x, W1)
    s1 = _make_segsum(N, H, NA, TOTCH, C0_SEG1)(g1, src2, dst2s)
    g2 = _make_enc2(N, H, NA, BM2)(degp2, s1, g1, b1r, W2)
    s2 = _make_segsum(N, H, NA, TOTCH, C0_SEG2)(g2, src2, dst2s)
    z2 = _make_fin(N, H, NA, BM2)(degp2, s2, g2, b2r)
    scores = _make_decode(N, H, TOTCH, C0_DEC)(z2, src2, dst2d)
    return scores[:E]
